# baseline (speedup 1.0000x reference)
import numpy as np

import jax
import jax.numpy as jnp
from jax import lax
from jax.experimental import pallas as pl
from jax.experimental.pallas import tpu as pltpu
from jax.experimental.pallas import tpu_sc as plsc

NUM_BAGS = 26
VOCAB = 100000
DIM = 32
BATCH = 4096
LPS = 50
W = NUM_BAGS * DIM

NC = 2
NS = 16
NW = NC * NS
SPW = BATCH // NW
CS = 16

VC = 1000
GROUP = 8


def _sel(nbags, perm):
    e = np.zeros((nbags, DIM, nbags * DIM), np.float32)
    for b in range(nbags):
        for p in range(DIM):
            e[b, perm[p], b * DIM + p] = 1.0
    return jnp.asarray(e, dtype=jnp.bfloat16)


_PERM = [(p % 2) * 16 + p // 2 for p in range(DIM)]


def _tx_body(x_ref, e8_ref, e2_ref, o_ref):
    xb = x_ref[...].astype(jnp.bfloat16)
    for g in range(3):
        acc = jnp.zeros((VC, GROUP * DIM), jnp.float32)
        for j in range(GROUP):
            acc += jax.lax.dot_general(
                xb[g * GROUP + j], e8_ref[j],
                (((1,), (0,)), ((), ())),
                preferred_element_type=jnp.float32,
            )
        o_ref[:, g * GROUP * DIM:(g + 1) * GROUP * DIM] = acc.astype(
            jnp.bfloat16)
    acc = jnp.zeros((VC, 2 * DIM), jnp.float32)
    for j in range(2):
        acc += jax.lax.dot_general(
            xb[3 * GROUP + j], e2_ref[j],
            (((1,), (0,)), ((), ())),
            preferred_element_type=jnp.float32,
        )
    o_ref[:, 3 * GROUP * DIM:] = acc.astype(jnp.bfloat16)


def _relayout_tc(tables, e8, e2):
    return pl.pallas_call(
        _tx_body,
        grid=(VOCAB // VC,),
        in_specs=[
            pl.BlockSpec((NUM_BAGS, VC, DIM), lambda i: (0, i, 0)),
            pl.BlockSpec((GROUP, DIM, GROUP * DIM), lambda i: (0, 0, 0)),
            pl.BlockSpec((2, DIM, 2 * DIM), lambda i: (0, 0, 0)),
        ],
        out_specs=pl.BlockSpec((VC, W), lambda i: (i, 0)),
        out_shape=jax.ShapeDtypeStruct((VOCAB, W), jnp.bfloat16),
    )(tables, e8, e2)


def _sc_body(idx_hbm, tab_hbm, out_hbm, idx_v, rows0, rows1, out_v, sem0, sem1):
    wid = lax.axis_index("s") * NC + lax.axis_index("c")
    base_sample = wid * SPW

    pltpu.sync_copy(idx_hbm.at[pl.ds(base_sample, SPW)], idx_v)

    bufs = (rows0, rows1)
    sems = (sem0, sem1)

    def start(t, parity):
        pltpu.async_copy(tab_hbm.at[idx_v.at[t]], bufs[parity], sems[parity])

    def wait(parity):
        pltpu.make_async_copy(
            tab_hbm.at[idx_v.at[0]], bufs[parity], sems[parity]
        ).wait()

    HI = jnp.int32(-65536)

    def reduce(t, buf):
        s_local = t & (CS - 1)

        def c_body(c, _):
            col = c * DIM

            def expand(l):
                xi = plsc.bitcast(buf[l, pl.ds(col, DIM)], jnp.int32)
                lo = plsc.bitcast(xi << 16, jnp.float32)
                hi = plsc.bitcast(xi & HI, jnp.float32)
                return lo, hi

            a0, a1 = expand(0)
            b0, b1 = expand(1)
            for l in range(2, LPS, 2):
                lo, hi = expand(l)
                a0 = a0 + lo
                a1 = a1 + hi
            for l in range(3, LPS, 2):
                lo, hi = expand(l)
                b0 = b0 + lo
                b1 = b1 + hi
            out_v[c, s_local, 0:16] = a0 + b0
            out_v[c, s_local, 16:32] = a1 + b1
            return 0

        lax.fori_loop(0, NUM_BAGS, c_body, 0)

    def flush(chunk):
        row0 = base_sample + chunk * CS
        pltpu.sync_copy(out_v, out_hbm.at[:, pl.ds(row0, CS), :])

    start(0, 0)

    def pair_body(p, _):
        t0 = p * 2
        t1 = t0 + 1
        start(t1, 1)
        wait(0)
        reduce(t0, rows0)

        @pl.when(t0 + 2 < SPW)
        def _prefetch0():
            start(t0 + 2, 0)

        wait(1)
        reduce(t1, rows1)

        @pl.when((t1 & (CS - 1)) == CS - 1)
        def _flush():
            flush(t1 >> 4)

        return 0

    lax.fori_loop(0, SPW // 2, pair_body, 0)


@jax.jit
def _run(idx, tables):
    e8 = _sel(GROUP, _PERM)
    e2 = _sel(2, _PERM)
    tab_t = _relayout_tc(tables, e8, e2)
    mesh = plsc.VectorSubcoreMesh(core_axis_name="c", subcore_axis_name="s")
    return pl.kernel(
        _sc_body,
        out_type=jax.ShapeDtypeStruct((NUM_BAGS, BATCH, DIM), jnp.float32),
        mesh=mesh,
        scratch_types=[
            pltpu.VMEM((SPW, LPS), jnp.int32),
            pltpu.VMEM((LPS, W), jnp.bfloat16),
            pltpu.VMEM((LPS, W), jnp.bfloat16),
            pltpu.VMEM((NUM_BAGS, CS, DIM), jnp.float32),
            pltpu.SemaphoreType.DMA,
            pltpu.SemaphoreType.DMA,
        ],
        compiler_params=pltpu.CompilerParams(
            use_tc_tiling_on_sc=False, needs_layout_passes=False),
    )(idx, tab_t)


def kernel(inputs, tables):
    return _run(inputs, tables).reshape(NUM_BAGS * BATCH, DIM)


# speedup vs baseline: 1.1640x; 1.1640x over previous
import jax
import jax.numpy as jnp
from jax import lax
from jax.experimental import pallas as pl
from jax.experimental.pallas import tpu as pltpu
from jax.experimental.pallas import tpu_sc as plsc

NUM_BAGS = 26
VOCAB = 100000
DIM = 32
BATCH = 4096
LPS = 50
W = NUM_BAGS * DIM

NC = 2
NS = 16
NW = NC * NS
SPW = BATCH // NW
CS = 16

VPT = VOCAB // NW
VCS = 25
NBLK = VPT // VCS


def _relayout_body(tab_hbm, out_hbm, in0, in1, o0, o1, si0, si1, so0, so1):
    wid = lax.axis_index("s") * NC + lax.axis_index("c")
    v_base = wid * VPT

    ins = (in0, in1)
    outs = (o0, o1)
    sis = (si0, si1)
    sos = (so0, so1)

    def start_in(blk, j):
        pltpu.async_copy(
            tab_hbm.at[:, pl.ds(v_base + blk * VCS, VCS), :], ins[j], sis[j]
        )

    def wait_in(j):
        pltpu.make_async_copy(
            tab_hbm.at[:, pl.ds(0, VCS), :], ins[j], sis[j]
        ).wait()

    def start_out(blk, j):
        pltpu.async_copy(
            outs[j], out_hbm.at[pl.ds(v_base + blk * VCS, VCS)], sos[j]
        )

    def wait_out(j):
        pltpu.make_async_copy(
            outs[j], out_hbm.at[pl.ds(0, VCS)], sos[j]
        ).wait()

    def repack(j):
        slab = ins[j]
        o = outs[j]

        def v_body(v, _):
            for b in range(NUM_BAGS):
                a = slab[b, v, 0:16]
                c = slab[b, v, 16:32]
                o[v, pl.ds(b * DIM, DIM)] = plsc.pack(
                    a, c, format=plsc.PackFormat.INTERLEAVED
                )
            return 0

        lax.fori_loop(0, VCS, v_body, 0)

    start_in(0, 0)
    start_in(1, 1)
    start_out(0, 0)
    start_out(1, 1)

    def pair_body(p, _):
        b0 = p * 2
        b1 = b0 + 1
        wait_in(0)
        wait_out(0)
        repack(0)
        start_out(b0, 0)

        @pl.when(b0 + 2 < NBLK)
        def _next0():
            start_in(b0 + 2, 0)

        wait_in(1)
        wait_out(1)
        repack(1)
        start_out(b1, 1)

        @pl.when(b1 + 2 < NBLK)
        def _next1():
            start_in(b1 + 2, 1)

        return 0

    lax.fori_loop(0, NBLK // 2, pair_body, 0)
    wait_in(0)
    wait_out(0)
    repack(0)
    start_out(NBLK - 1, 0)
    wait_out(0)
    wait_out(1)


def _sc_body(idx_hbm, tab_hbm, out_hbm, idx_v, rows0, rows1, out_v, sem0, sem1):
    wid = lax.axis_index("s") * NC + lax.axis_index("c")
    base_sample = wid * SPW

    pltpu.sync_copy(idx_hbm.at[pl.ds(base_sample, SPW)], idx_v)

    bufs = (rows0, rows1)
    sems = (sem0, sem1)

    def start(t, parity):
        pltpu.async_copy(tab_hbm.at[idx_v.at[t]], bufs[parity], sems[parity])

    def wait(parity):
        pltpu.make_async_copy(
            tab_hbm.at[idx_v.at[0]], bufs[parity], sems[parity]
        ).wait()

    HI = jnp.int32(-65536)

    def reduce(t, buf):
        s_local = t & (CS - 1)

        def c_body(c, _):
            col = c * DIM

            def expand(l):
                xi = plsc.bitcast(buf[l, pl.ds(col, DIM)], jnp.int32)
                lo = plsc.bitcast(xi << 16, jnp.float32)
                hi = plsc.bitcast(xi & HI, jnp.float32)
                return lo, hi

            a0, a1 = expand(0)
            b0, b1 = expand(1)
            for l in range(2, LPS, 2):
                lo, hi = expand(l)
                a0 = a0 + lo
                a1 = a1 + hi
            for l in range(3, LPS, 2):
                lo, hi = expand(l)
                b0 = b0 + lo
                b1 = b1 + hi
            out_v[c, s_local, 0:16] = a0 + b0
            out_v[c, s_local, 16:32] = a1 + b1
            return 0

        lax.fori_loop(0, NUM_BAGS, c_body, 0)

    def flush(chunk):
        row0 = base_sample + chunk * CS
        pltpu.sync_copy(out_v, out_hbm.at[:, pl.ds(row0, CS), :])

    start(0, 0)

    def pair_body(p, _):
        t0 = p * 2
        t1 = t0 + 1
        start(t1, 1)
        wait(0)
        reduce(t0, rows0)

        @pl.when(t0 + 2 < SPW)
        def _prefetch0():
            start(t0 + 2, 0)

        wait(1)
        reduce(t1, rows1)

        @pl.when((t1 & (CS - 1)) == CS - 1)
        def _flush():
            flush(t1 >> 4)

        return 0

    lax.fori_loop(0, SPW // 2, pair_body, 0)


@jax.jit
def _run(idx, tables):
    mesh = plsc.VectorSubcoreMesh(core_axis_name="c", subcore_axis_name="s")
    params = pltpu.CompilerParams(
        use_tc_tiling_on_sc=False, needs_layout_passes=False)

    tab_t = pl.kernel(
        _relayout_body,
        out_type=jax.ShapeDtypeStruct((VOCAB, W), jnp.bfloat16),
        mesh=mesh,
        scratch_types=[
            pltpu.VMEM((NUM_BAGS, VCS, DIM), jnp.float32),
            pltpu.VMEM((NUM_BAGS, VCS, DIM), jnp.float32),
            pltpu.VMEM((VCS, W), jnp.bfloat16),
            pltpu.VMEM((VCS, W), jnp.bfloat16),
            pltpu.SemaphoreType.DMA,
            pltpu.SemaphoreType.DMA,
            pltpu.SemaphoreType.DMA,
            pltpu.SemaphoreType.DMA,
        ],
        compiler_params=params,
    )(tables)

    return pl.kernel(
        _sc_body,
        out_type=jax.ShapeDtypeStruct((NUM_BAGS, BATCH, DIM), jnp.float32),
        mesh=mesh,
        scratch_types=[
            pltpu.VMEM((SPW, LPS), jnp.int32),
            pltpu.VMEM((LPS, W), jnp.bfloat16),
            pltpu.VMEM((LPS, W), jnp.bfloat16),
            pltpu.VMEM((NUM_BAGS, CS, DIM), jnp.float32),
            pltpu.SemaphoreType.DMA,
            pltpu.SemaphoreType.DMA,
        ],
        compiler_params=params,
    )(idx, tab_t)


def kernel(inputs, tables):
    return _run(inputs, tables).reshape(NUM_BAGS * BATCH, DIM)
